# baseline (speedup 1.0000x reference)
import functools

import jax
import jax.numpy as jnp
from jax import lax
from jax.experimental import pallas as pl
from jax.experimental.pallas import tpu as pltpu
from jax.experimental.pallas import tpu_sc as plsc

NUM_ROWS = 1000000
DIM = 64
WORDS = DIM // 4

NC, NS, L = 2, 16, 16
NW = NC * NS

BLK = 128


def _body(idx_hbm, qw_hbm, s_hbm, zp_hbm, out_hbm,
          idx_v, rows_v, s_v, zp_v, out_v, sem, nblk):
    wid = lax.axis_index("s") * NC + lax.axis_index("c")
    base_blk = wid * nblk

    pltpu.sync_copy(idx_hbm.at[pl.ds(base_blk * BLK, nblk * BLK)], idx_v)

    lane = lax.iota(jnp.int32, L)

    def block(b, _):
        iref = idx_v.at[pl.ds(b * BLK, BLK)]
        cp_q = pltpu.async_copy(qw_hbm.at[iref], rows_v, sem)
        cp_s = pltpu.async_copy(s_hbm.at[iref], s_v, sem)
        cp_z = pltpu.async_copy(zp_hbm.at[iref], zp_v, sem)
        cp_q.wait()
        cp_s.wait()
        cp_z.wait()

        def group(g, _):
            r0 = g * L
            rvec = lane + r0
            s_vec = s_v[pl.ds(r0, L)]
            zp_vec = zp_v[pl.ds(r0, L)]
            c_vec = s_vec * zp_vec
            for k in range(WORDS):
                wk = plsc.load_gather(rows_v, [rvec, jnp.full((L,), k, jnp.int32)])
                wu = plsc.bitcast(wk, jnp.uint32)
                for j in range(4):
                    if j == 0:
                        byte = wu & 0xFF
                    elif j == 3:
                        byte = wu >> 24
                    else:
                        byte = (wu >> (8 * j)) & 0xFF
                    y = byte.astype(jnp.float32) * s_vec - c_vec
                    col = jnp.full((L,), 4 * k + j, jnp.int32)
                    plsc.store_scatter(out_v, [rvec, col], y)
            return _

        lax.fori_loop(0, BLK // L, group, None)
        pltpu.sync_copy(out_v, out_hbm.at[pl.ds((base_blk + b) * BLK, BLK)])
        return _

    lax.fori_loop(0, nblk, block, None)


@functools.partial(jax.jit, static_argnames=())
def _run(idx_flat, qw_i32, scales, zero_points):
    total = idx_flat.shape[0]
    nblk = total // (NW * BLK)

    mesh = plsc.VectorSubcoreMesh(core_axis_name="c", subcore_axis_name="s")
    out = pl.kernel(
        functools.partial(_body, nblk=nblk),
        out_type=jax.ShapeDtypeStruct((total, DIM), jnp.float32),
        mesh=mesh,
        compiler_params=pltpu.CompilerParams(
            needs_layout_passes=False, use_tc_tiling_on_sc=False),
        scratch_types=[
            pltpu.VMEM((nblk * BLK,), jnp.int32),
            pltpu.VMEM((BLK, WORDS), jnp.int32),
            pltpu.VMEM((BLK,), jnp.float32),
            pltpu.VMEM((BLK,), jnp.float32),
            pltpu.VMEM((BLK, DIM), jnp.float32),
            pltpu.SemaphoreType.DMA,
        ],
    )(idx_flat, qw_i32, scales, zero_points)
    return out


def kernel(indices, qweight, scales, zero_points):
    batch, hist = indices.shape
    idx_flat = indices.reshape(batch * hist)
    qw_i32 = lax.bitcast_convert_type(
        qweight.reshape(NUM_ROWS, WORDS, 4), jnp.int32)
    out = _run(idx_flat, qw_i32, scales, zero_points)
    return out.reshape(batch, hist, DIM)


# speedup vs baseline: 1.8147x; 1.8147x over previous
import functools

import jax
import jax.numpy as jnp
from jax import lax
from jax.experimental import pallas as pl
from jax.experimental.pallas import tpu as pltpu
from jax.experimental.pallas import tpu_sc as plsc

NUM_ROWS = 1000000
DIM = 64
WORDS = DIM // 4

NC, NS, L = 2, 16, 16
NW = NC * NS

BLK = 128


def _body(idx_hbm, qw_hbm, s_hbm, zp_hbm, out_hbm,
          idx_v, rows_v, s_v, zp_v, out_v, sem, nblk):
    wid = lax.axis_index("s") * NC + lax.axis_index("c")
    base_blk = wid * nblk

    pltpu.sync_copy(idx_hbm.at[pl.ds(base_blk * BLK, nblk * BLK)], idx_v)

    lane = lax.iota(jnp.int32, L)

    def block(b, _):
        iref = idx_v.at[pl.ds(b * BLK, BLK)]
        cp_q = pltpu.async_copy(qw_hbm.at[iref], rows_v, sem)
        cp_s = pltpu.async_copy(s_hbm.at[iref], s_v, sem)
        cp_z = pltpu.async_copy(zp_hbm.at[iref], zp_v, sem)
        cp_q.wait()
        cp_s.wait()
        cp_z.wait()

        def row(r, _):
            w64 = rows_v[r, :]
            wu = plsc.bitcast(w64, jnp.uint32)
            rfull = jnp.full((L,), r, jnp.int32)
            sb = plsc.load_gather(s_v, [rfull])
            zb = plsc.load_gather(zp_v, [rfull])
            cb = sb * zb
            for j in range(4):
                if j == 0:
                    byte = wu & 0xFF
                elif j == 3:
                    byte = wu >> 24
                else:
                    byte = (wu >> (8 * j)) & 0xFF
                y = byte.astype(jnp.float32) * sb - cb
                plsc.store_scatter(out_v, [rfull, 4 * lane + j], y)
            return _

        lax.fori_loop(0, BLK, row, None, unroll=2)
        pltpu.sync_copy(out_v, out_hbm.at[pl.ds((base_blk + b) * BLK, BLK)])
        return _

    lax.fori_loop(0, nblk, block, None)


@functools.partial(jax.jit, static_argnames=())
def _run(idx_flat, qw_i32, scales, zero_points):
    total = idx_flat.shape[0]
    nblk = total // (NW * BLK)

    mesh = plsc.VectorSubcoreMesh(core_axis_name="c", subcore_axis_name="s")
    out = pl.kernel(
        functools.partial(_body, nblk=nblk),
        out_type=jax.ShapeDtypeStruct((total, DIM), jnp.float32),
        mesh=mesh,
        compiler_params=pltpu.CompilerParams(
            needs_layout_passes=False, use_tc_tiling_on_sc=False),
        scratch_types=[
            pltpu.VMEM((nblk * BLK,), jnp.int32),
            pltpu.VMEM((BLK, DIM), jnp.uint8),
            pltpu.VMEM((BLK,), jnp.float32),
            pltpu.VMEM((BLK,), jnp.float32),
            pltpu.VMEM((BLK, DIM), jnp.float32),
            pltpu.SemaphoreType.DMA,
        ],
    )(idx_flat, qw_i32, scales, zero_points)
    return out


def kernel(indices, qweight, scales, zero_points):
    batch, hist = indices.shape
    idx_flat = indices.reshape(batch * hist)
    out = _run(idx_flat, qweight, scales, zero_points)
    return out.reshape(batch, hist, DIM)
